# Initial kernel scaffold; baseline (speedup 1.0000x reference)
import functools

import jax
import jax.numpy as jnp
from jax import lax
from jax.experimental import pallas as pl
from jax.experimental.pallas import tpu as pltpu
from jax.experimental.pallas import tpu_sc as plsc

N = 10000
D = 128
E = 320000

NC = 2
NS = 16
NW = NC * NS
CHUNK = 128
NCH = 80
EPW = CHUNK * NCH
E_PAD = EPW * NW
NP = 10240
RPT = NP // NS
BR = 512
CW = 8


def _sc_body(with_cnt, *refs):
    if with_cnt:
        (x_hbm, src_hbm, dst_hbm, z_hbm, z8_hbm, ones_hbm,
         p_hbm, cnt_hbm,
         sidx, didx, rows0, rows1, ones_v, agg_sh, cnt_sh, sem0, sem1) = refs
    else:
        (x_hbm, src_hbm, dst_hbm, z_hbm,
         p_hbm,
         sidx, didx, rows0, rows1, agg_sh, sem0, sem1) = refs

    c = lax.axis_index("c")
    s = lax.axis_index("s")
    wid = c * NS + s
    row0 = s * RPT

    pltpu.sync_copy(src_hbm.at[wid], sidx)
    pltpu.sync_copy(dst_hbm.at[wid], didx)
    pltpu.sync_copy(z_hbm, agg_sh.at[pl.ds(row0, RPT)])
    if with_cnt:
        pltpu.sync_copy(z8_hbm, cnt_sh.at[pl.ds(row0, RPT)])
        pltpu.sync_copy(ones_hbm, ones_v)
    plsc.subcore_barrier()

    bufs = (rows0, rows1)
    sems = (sem0, sem1)

    def start(j, b):
        pltpu.make_async_copy(x_hbm.at[sidx.at[j]], bufs[b], sems[b]).start()

    def finish(j, b):
        pltpu.make_async_copy(x_hbm.at[sidx.at[j]], bufs[b], sems[b]).wait()
        pltpu.sync_copy(bufs[b], agg_sh.at[didx.at[j]], add=True)
        if with_cnt:
            pltpu.sync_copy(ones_v, cnt_sh.at[didx.at[j]], add=True)

    start(0, 0)

    def g_body(g, carry):
        j0 = 2 * g
        start(j0 + 1, 1)
        finish(j0, 0)

        @pl.when(g + 1 < NCH // 2)
        def _():
            start(j0 + 2, 0)

        finish(j0 + 1, 1)
        return carry

    lax.fori_loop(0, NCH // 2, g_body, 0)

    plsc.subcore_barrier()
    pltpu.sync_copy(agg_sh.at[pl.ds(row0, RPT)],
                    p_hbm.at[c, pl.ds(row0, RPT)])
    if with_cnt:
        pltpu.sync_copy(cnt_sh.at[pl.ds(row0, RPT)],
                        cnt_hbm.at[c, pl.ds(row0, RPT)])


def _make_sc(with_cnt):
    mesh = plsc.VectorSubcoreMesh(core_axis_name="c", subcore_axis_name="s")
    outs = [jax.ShapeDtypeStruct((NC, NP, D), jnp.float32)]
    scratch = [
        pltpu.VMEM((NCH, CHUNK), jnp.int32),
        pltpu.VMEM((NCH, CHUNK), jnp.int32),
        pltpu.VMEM((CHUNK, D), jnp.float32),
        pltpu.VMEM((CHUNK, D), jnp.float32),
    ]
    if with_cnt:
        outs.append(jax.ShapeDtypeStruct((NC, NP, CW), jnp.float32))
        scratch.append(pltpu.VMEM((CHUNK, CW), jnp.float32))
    scratch.append(pltpu.VMEM_SHARED((NP, D), jnp.float32))
    if with_cnt:
        scratch.append(pltpu.VMEM_SHARED((NP, CW), jnp.float32))
    scratch += [pltpu.SemaphoreType.DMA, pltpu.SemaphoreType.DMA]
    return pl.kernel(
        functools.partial(_sc_body, with_cnt),
        out_type=outs if with_cnt else outs[0],
        scratch_types=scratch,
        mesh=mesh,
    )


def _tc_body(relu, p_ref, cnt_ref, x_ref, wl_ref, wr_ref, b_ref, o_ref):
    cnt = cnt_ref[0] + cnt_ref[1]
    inv = 1.0 / jnp.maximum(cnt[:, 0:1], 1.0)
    mean = (p_ref[0] + p_ref[1]) * inv
    acc = jnp.dot(mean, wl_ref[...], preferred_element_type=jnp.float32)
    acc = acc + jnp.dot(x_ref[...], wr_ref[...],
                        preferred_element_type=jnp.float32)
    acc = acc + b_ref[...]
    if relu:
        acc = jnp.maximum(acc, 0.0)
    o_ref[...] = acc


def _make_tc(relu):
    return pl.pallas_call(
        functools.partial(_tc_body, relu),
        grid=(NP // BR,),
        in_specs=[
            pl.BlockSpec((NC, BR, D), lambda i: (0, i, 0)),
            pl.BlockSpec((NC, BR, CW), lambda i: (0, i, 0)),
            pl.BlockSpec((BR, D), lambda i: (i, 0)),
            pl.BlockSpec((D, D), lambda i: (0, 0)),
            pl.BlockSpec((D, D), lambda i: (0, 0)),
            pl.BlockSpec((1, D), lambda i: (0, 0)),
        ],
        out_specs=pl.BlockSpec((BR, D), lambda i: (i, 0)),
        out_shape=jax.ShapeDtypeStruct((NP, D), jnp.float32),
    )


_sc_agg_cnt = _make_sc(True)
_sc_agg = _make_sc(False)
_tc_relu = _make_tc(True)
_tc_lin = _make_tc(False)


def kernel(x, edge_index, W1l, b1l, W1r, W2l, b2l, W2r):
    pad = E_PAD - E
    src = jnp.concatenate([edge_index[0], jnp.zeros((pad,), jnp.int32)])
    dst = jnp.concatenate([edge_index[1], jnp.full((pad,), N, jnp.int32)])
    src3 = src.reshape(NW, NCH, CHUNK)
    dst3 = dst.reshape(NW, NCH, CHUNK)

    zeros = jnp.zeros((RPT, D), jnp.float32)
    zeros8 = jnp.zeros((RPT, CW), jnp.float32)
    ones = jnp.ones((CHUNK, CW), jnp.float32)
    x_pad = jnp.concatenate([x, jnp.zeros((NP - N, D), jnp.float32)])

    p1, cnt = _sc_agg_cnt(x_pad, src3, dst3, zeros, zeros8, ones)
    h = _tc_relu(p1, cnt, x_pad, W1l.T, W1r.T, b1l.reshape(1, D))
    p2 = _sc_agg(h, src3, dst3, zeros)
    out = _tc_lin(p2, cnt, h, W2l.T, W2r.T, b2l.reshape(1, D))
    return out[:N]


# speedup vs baseline: 3.8297x; 3.8297x over previous
import functools

import jax
import jax.numpy as jnp
from jax import lax
from jax.experimental import pallas as pl
from jax.experimental.pallas import tpu as pltpu
from jax.experimental.pallas import tpu_sc as plsc

N = 10000
D = 128
DH = D // 2
E = 320000

NC = 2
NS = 16
NW = NC * NS
CHUNK = 128
NCH = 80
EPW = CHUNK * NCH
E_PAD = EPW * NW
NP = 10240
RPT = NP // NS
BR = 512
CW = 8


def _sc_body(with_cnt, *refs):
    if with_cnt:
        (x0_hbm, x1_hbm, src_hbm, dst_hbm, z_hbm, z8_hbm, ones_hbm,
         p_hbm, cnt_hbm,
         sidx, didx, rows0, rows1, ones_v, agg_sh, cnt_sh, sem0, sem1) = refs
    else:
        (x0_hbm, x1_hbm, src_hbm, dst_hbm, z_hbm,
         p_hbm,
         sidx, didx, rows0, rows1, agg_sh, sem0, sem1) = refs

    c = lax.axis_index("c")
    s = lax.axis_index("s")
    wid = c * NS + s
    row0 = s * RPT

    pltpu.sync_copy(src_hbm.at[wid], sidx)
    pltpu.sync_copy(dst_hbm.at[wid], didx)
    if with_cnt:
        pltpu.sync_copy(ones_hbm, ones_v)

    bufs = (rows0, rows1)
    sems = (sem0, sem1)

    for half, xh_hbm in enumerate((x0_hbm, x1_hbm)):
        do_cnt = with_cnt and half == 0

        pltpu.sync_copy(z_hbm, agg_sh.at[pl.ds(row0, RPT)])
        if do_cnt:
            pltpu.sync_copy(z8_hbm, cnt_sh.at[pl.ds(row0, RPT)])
        plsc.subcore_barrier()

        def start(j, b):
            pltpu.make_async_copy(
                xh_hbm.at[sidx.at[j]], bufs[b], sems[b]).start()

        def finish(j, b):
            pltpu.make_async_copy(
                xh_hbm.at[sidx.at[j]], bufs[b], sems[b]).wait()
            pltpu.sync_copy(bufs[b], agg_sh.at[didx.at[j]], add=True)
            if do_cnt:
                pltpu.sync_copy(ones_v, cnt_sh.at[didx.at[j]], add=True)

        start(0, 0)

        def g_body(g, carry):
            j0 = 2 * g
            start(j0 + 1, 1)
            finish(j0, 0)

            @pl.when(g + 1 < NCH // 2)
            def _():
                start(j0 + 2, 0)

            finish(j0 + 1, 1)
            return carry

        lax.fori_loop(0, NCH // 2, g_body, 0)

        plsc.subcore_barrier()
        pltpu.sync_copy(agg_sh.at[pl.ds(row0, RPT)],
                        p_hbm.at[c, half, pl.ds(row0, RPT)])
        if do_cnt:
            pltpu.sync_copy(cnt_sh.at[pl.ds(row0, RPT)],
                            cnt_hbm.at[c, pl.ds(row0, RPT)])


def _make_sc(with_cnt):
    mesh = plsc.VectorSubcoreMesh(core_axis_name="c", subcore_axis_name="s")
    outs = [jax.ShapeDtypeStruct((NC, 2, NP, DH), jnp.float32)]
    scratch = [
        pltpu.VMEM((NCH, CHUNK), jnp.int32),
        pltpu.VMEM((NCH, CHUNK), jnp.int32),
        pltpu.VMEM((CHUNK, DH), jnp.float32),
        pltpu.VMEM((CHUNK, DH), jnp.float32),
    ]
    if with_cnt:
        outs.append(jax.ShapeDtypeStruct((NC, NP, CW), jnp.float32))
        scratch.append(pltpu.VMEM((CHUNK, CW), jnp.float32))
    scratch.append(pltpu.VMEM_SHARED((NP, DH), jnp.float32))
    if with_cnt:
        scratch.append(pltpu.VMEM_SHARED((NP, CW), jnp.float32))
    scratch += [pltpu.SemaphoreType.DMA, pltpu.SemaphoreType.DMA]
    return pl.kernel(
        functools.partial(_sc_body, with_cnt),
        out_type=outs if with_cnt else outs[0],
        scratch_types=scratch,
        mesh=mesh,
        compiler_params=pltpu.CompilerParams(use_tc_tiling_on_sc=False),
    )


def _tc_body(relu, p_ref, cnt_ref, x0_ref, x1_ref, wl_ref, wr_ref, b_ref,
             *o_refs):
    cnt = cnt_ref[0] + cnt_ref[1]
    inv = 1.0 / jnp.maximum(cnt[:, 0:1], 1.0)
    f32 = jnp.float32
    acc = jnp.dot((p_ref[0, 0] + p_ref[1, 0]) * inv, wl_ref[0:DH, :],
                  preferred_element_type=f32)
    acc = acc + jnp.dot((p_ref[0, 1] + p_ref[1, 1]) * inv, wl_ref[DH:D, :],
                        preferred_element_type=f32)
    acc = acc + jnp.dot(x0_ref[...], wr_ref[0:DH, :],
                        preferred_element_type=f32)
    acc = acc + jnp.dot(x1_ref[...], wr_ref[DH:D, :],
                        preferred_element_type=f32)
    acc = acc + b_ref[...]
    if relu:
        acc = jnp.maximum(acc, 0.0)
    if len(o_refs) == 2:
        o_refs[0][...] = acc[:, 0:DH]
        o_refs[1][...] = acc[:, DH:D]
    else:
        o_refs[0][...] = acc


def _make_tc(relu, split_out):
    if split_out:
        out_shape = [jax.ShapeDtypeStruct((NP, DH), jnp.float32)] * 2
        out_specs = [pl.BlockSpec((BR, DH), lambda i: (i, 0))] * 2
    else:
        out_shape = jax.ShapeDtypeStruct((NP, D), jnp.float32)
        out_specs = pl.BlockSpec((BR, D), lambda i: (i, 0))
    return pl.pallas_call(
        functools.partial(_tc_body, relu),
        grid=(NP // BR,),
        in_specs=[
            pl.BlockSpec((NC, 2, BR, DH), lambda i: (0, 0, i, 0)),
            pl.BlockSpec((NC, BR, CW), lambda i: (0, i, 0)),
            pl.BlockSpec((BR, DH), lambda i: (i, 0)),
            pl.BlockSpec((BR, DH), lambda i: (i, 0)),
            pl.BlockSpec((D, D), lambda i: (0, 0)),
            pl.BlockSpec((D, D), lambda i: (0, 0)),
            pl.BlockSpec((1, D), lambda i: (0, 0)),
        ],
        out_specs=out_specs,
        out_shape=out_shape,
    )


_sc_agg_cnt = _make_sc(True)
_sc_agg = _make_sc(False)
_tc_l1 = _make_tc(True, True)
_tc_l2 = _make_tc(False, False)


def kernel(x, edge_index, W1l, b1l, W1r, W2l, b2l, W2r):
    pad = E_PAD - E
    src = jnp.concatenate([edge_index[0], jnp.zeros((pad,), jnp.int32)])
    dst = jnp.concatenate([edge_index[1], jnp.full((pad,), N, jnp.int32)])
    src3 = src.reshape(NW, NCH, CHUNK)
    dst3 = dst.reshape(NW, NCH, CHUNK)

    zeros = jnp.zeros((RPT, DH), jnp.float32)
    zeros8 = jnp.zeros((RPT, CW), jnp.float32)
    ones = jnp.ones((CHUNK, CW), jnp.float32)
    rpad = jnp.zeros((NP - N, DH), jnp.float32)
    x0 = jnp.concatenate([x[:, 0:DH], rpad])
    x1 = jnp.concatenate([x[:, DH:D], rpad])

    p1, cnt = _sc_agg_cnt(x0, x1, src3, dst3, zeros, zeros8, ones)
    h0, h1 = _tc_l1(p1, cnt, x0, x1, W1l.T, W1r.T, b1l.reshape(1, D))
    p2 = _sc_agg(h0, h1, src3, dst3, zeros)
    out = _tc_l2(p2, cnt, h0, h1, W2l.T, W2r.T, b2l.reshape(1, D))
    return out[:N]
